# baseline (speedup 1.0000x reference)
import jax
import jax.numpy as jnp
from jax import lax
from jax.experimental import pallas as pl
from jax.experimental.pallas import tpu as pltpu

_HAAR_LL_SCALE = 0.35355339059327373


def _haar_ll_kernel(x_ref, o_ref):
    td, hq, w = o_ref.shape
    de, do = pl.ds(0, td, 2), pl.ds(1, td, 2)
    h_even = (
        x_ref[de, pl.ds(0, hq, 4), :]
        + x_ref[de, pl.ds(1, hq, 4), :]
        + x_ref[do, pl.ds(0, hq, 4), :]
        + x_ref[do, pl.ds(1, hq, 4), :]
    ).reshape(td * hq, w)
    h_odd = (
        x_ref[de, pl.ds(2, hq, 4), :]
        + x_ref[de, pl.ds(3, hq, 4), :]
        + x_ref[do, pl.ds(2, hq, 4), :]
        + x_ref[do, pl.ds(3, hq, 4), :]
    ).reshape(td * hq, w)
    r = lax.broadcasted_iota(jnp.int32, (w, w), 0)
    c = lax.broadcasted_iota(jnp.int32, (w, w), 1)
    sel = r // 2
    p_lo = (sel == c).astype(jnp.float32)
    p_hi = (sel == c - w // 2).astype(jnp.float32)
    m = jnp.dot(h_even, p_lo, preferred_element_type=jnp.float32) + jnp.dot(
        h_odd, p_hi, preferred_element_type=jnp.float32
    )
    m = m * jnp.asarray(_HAAR_LL_SCALE, dtype=jnp.float32)
    o_ref[...] = m.reshape(td, hq, w).astype(o_ref.dtype)


def kernel(x):
    B, C, D, H, W = x.shape
    n = B * C * D
    xf = x.reshape(n, H, W)
    TD = min(64, n // 2)
    grid = (n // 2) // TD
    out = pl.pallas_call(
        _haar_ll_kernel,
        grid=(grid,),
        in_specs=[pl.BlockSpec((2 * TD, H, W), lambda i: (i, 0, 0))],
        out_specs=pl.BlockSpec((TD, H // 4, W), lambda i: (i, 0, 0)),
        out_shape=jax.ShapeDtypeStruct((n // 2, H // 4, W), x.dtype),
        compiler_params=pltpu.CompilerParams(
            dimension_semantics=("parallel",),
            vmem_limit_bytes=100 * 1024 * 1024,
        ),
        name="haar3d_ll",
    )(xf)
    out = out.reshape(B, C, D // 2, H // 2, W // 2)
    if C == 1:
        out = out.squeeze(1)
    return out


# speedup vs baseline: 1.4073x; 1.4073x over previous
import jax
import jax.numpy as jnp
from jax import lax
from jax.experimental import pallas as pl
from jax.experimental.pallas import tpu as pltpu

_HAAR_LL_SCALE = 0.35355339059327373


def _haar_ll_kernel(x_ref, o_ref):
    td, hh, hw = o_ref.shape
    h = (
        x_ref[pl.ds(0, td, 2), pl.ds(0, hh, 2), :]
        + x_ref[pl.ds(0, td, 2), pl.ds(1, hh, 2), :]
        + x_ref[pl.ds(1, td, 2), pl.ds(0, hh, 2), :]
        + x_ref[pl.ds(1, td, 2), pl.ds(1, hh, 2), :]
    )
    r = lax.broadcasted_iota(jnp.int32, (2 * hw, hw), 0)
    c = lax.broadcasted_iota(jnp.int32, (2 * hw, hw), 1)
    p = (r // 2 == c).astype(jnp.float32)
    m = jnp.dot(
        h.reshape(td * hh, 2 * hw), p, preferred_element_type=jnp.float32
    )
    m = m * jnp.asarray(_HAAR_LL_SCALE, dtype=jnp.float32)
    o_ref[...] = m.reshape(td, hh, hw).astype(o_ref.dtype)


def kernel(x):
    B, C, D, H, W = x.shape
    n = B * C * D
    xf = x.reshape(n, H, W)
    TD = min(64, n // 2)
    grid = (n // 2) // TD
    out = pl.pallas_call(
        _haar_ll_kernel,
        grid=(grid,),
        in_specs=[pl.BlockSpec((2 * TD, H, W), lambda i: (i, 0, 0))],
        out_specs=pl.BlockSpec((TD, H // 2, W // 2), lambda i: (i, 0, 0)),
        out_shape=jax.ShapeDtypeStruct((n // 2, H // 2, W // 2), x.dtype),
        compiler_params=pltpu.CompilerParams(
            dimension_semantics=("parallel",),
            vmem_limit_bytes=100 * 1024 * 1024,
        ),
        name="haar3d_ll",
    )(xf)
    out = out.reshape(B, C, D // 2, H // 2, W // 2)
    if C == 1:
        out = out.squeeze(1)
    return out
